# baseline (speedup 1.0000x reference)
import functools

import jax
import jax.numpy as jnp
from jax import lax
from jax.experimental import pallas as pl
from jax.experimental.pallas import tpu as pltpu
from jax.experimental.pallas import tpu_sc as plsc

N = 10000
F = 128
HID = 128
OUT = 3
G = 64
E = 320000

NC = 2
NS = 16
NW = NC * NS
L = 16

CHUNK = 64
NP = 10240
EPW = NP
E_PAD = NW * EPW
TOT_CHUNKS = E_PAD // CHUNK
C0 = TOT_CHUNKS // NS
SLAB = 40
RPT = NP // NS

BLK = 2048
NBLK = NP // BLK


DCH = 512


def _sc_deg_body(col_hbm, parts_hbm, col_v, deg_v):
    c = lax.axis_index("c")
    s = lax.axis_index("s")
    wid = s * NC + c

    def zero(i, _):
        deg_v[pl.ds(i * L, L)] = jnp.zeros((L,), jnp.float32)
        return 0

    lax.fori_loop(0, NP // L, zero, 0)

    ones = jnp.ones((L,), jnp.float32)

    def outer(t, _):
        pltpu.sync_copy(col_hbm.at[wid, pl.ds(t * DCH, DCH)], col_v)

        def body(k, _):
            idx = col_v[pl.ds(k * L, L)]
            plsc.addupdate_scatter(deg_v, [idx], ones)
            return 0

        lax.fori_loop(0, DCH // L, body, 0)
        return 0

    lax.fori_loop(0, EPW // DCH, outer, 0)
    pltpu.sync_copy(deg_v, parts_hbm.at[wid])


def _sc_degree(col_flat):
    mesh = plsc.VectorSubcoreMesh(core_axis_name="c", subcore_axis_name="s")
    return pl.kernel(
        _sc_deg_body,
        out_type=jax.ShapeDtypeStruct((NW, NP), jnp.float32),
        mesh=mesh,
        compiler_params=pltpu.CompilerParams(needs_layout_passes=False),
        scratch_types=[
            pltpu.VMEM((DCH,), jnp.int32),
            pltpu.VMEM((NP,), jnp.float32),
        ],
    )(col_flat)


def _tc_scale_body(x_ref, w_ref, parts_ref, y_ref, dinv_ref):
    deg = jnp.sum(parts_ref[...], axis=0) + 1.0
    dinv = lax.rsqrt(deg).reshape(BLK, 1)
    xw = jnp.dot(x_ref[...], w_ref[...], preferred_element_type=jnp.float32)
    y_ref[...] = xw * dinv
    dinv_ref[...] = dinv


def _tc_scale(x_p, W1, parts):
    return pl.pallas_call(
        _tc_scale_body,
        grid=(NBLK,),
        in_specs=[
            pl.BlockSpec((BLK, F), lambda i: (i, 0)),
            pl.BlockSpec((F, HID), lambda i: (0, 0)),
            pl.BlockSpec((NW, BLK), lambda i: (0, i)),
        ],
        out_specs=[
            pl.BlockSpec((BLK, HID), lambda i: (i, 0)),
            pl.BlockSpec((BLK, 1), lambda i: (i, 0)),
        ],
        out_shape=[
            jax.ShapeDtypeStruct((NP, HID), jnp.float32),
            jax.ShapeDtypeStruct((NP, 1), jnp.float32),
        ],
    )(x_p, W1, parts)


def _sc_scatter_body(y_hbm, row_hbm, col_hbm, z_hbm, out_hbm,
                     row_v, col_v, buf0, buf1, gsem0, gsem1, ssem0, ssem1,
                     acc):
    c = lax.axis_index("c")
    s = lax.axis_index("s")
    base = s * C0
    nslab = jnp.where(c == 0, C0 // SLAB, 0)

    @pl.when(c == 0)
    def _():
        pltpu.sync_copy(z_hbm, acc.at[pl.ds(s * RPT, RPT)])

    plsc.subcore_barrier()

    pairs = SLAB // 2

    def body(j, _):
        j0 = 2 * j
        j1 = 2 * j + 1
        pltpu.make_async_copy(y_hbm.at[row_v.at[j0]], buf0, gsem0).wait()
        pltpu.async_copy(buf0, acc.at[col_v.at[j0]], ssem0, add=True)
        pltpu.make_async_copy(y_hbm.at[row_v.at[j1]], buf1, gsem1).wait()
        pltpu.async_copy(buf1, acc.at[col_v.at[j1]], ssem1, add=True)

        @pl.when(j < pairs - 1)
        def _():
            pltpu.make_async_copy(buf0, acc.at[col_v.at[j0]], ssem0).wait()
            pltpu.async_copy(y_hbm.at[row_v.at[j0 + 2]], buf0, gsem0)
            pltpu.make_async_copy(buf1, acc.at[col_v.at[j1]], ssem1).wait()
            pltpu.async_copy(y_hbm.at[row_v.at[j1 + 2]], buf1, gsem1)

        return 0

    def slab_body(t, _):
        off = base + t * SLAB
        pltpu.sync_copy(row_hbm.at[pl.ds(off, SLAB)], row_v)
        pltpu.sync_copy(col_hbm.at[pl.ds(off, SLAB)], col_v)
        pltpu.async_copy(y_hbm.at[row_v.at[0]], buf0, gsem0)
        pltpu.async_copy(y_hbm.at[row_v.at[1]], buf1, gsem1)
        lax.fori_loop(0, pairs, body, 0)
        pltpu.make_async_copy(buf0, acc.at[col_v.at[0]], ssem0).wait()
        pltpu.make_async_copy(buf1, acc.at[col_v.at[1]], ssem1).wait()
        return 0

    lax.fori_loop(0, nslab, slab_body, 0)
    plsc.subcore_barrier()

    @pl.when(c == 0)
    def _():
        pltpu.sync_copy(acc.at[pl.ds(s * RPT, RPT)],
                        out_hbm.at[pl.ds(s * RPT, RPT)])


def _sc_scatter(y, row_p, col_p, zeros_rpt):
    mesh = plsc.VectorSubcoreMesh(core_axis_name="c", subcore_axis_name="s")
    return pl.kernel(
        _sc_scatter_body,
        out_type=jax.ShapeDtypeStruct((NP, HID), jnp.float32),
        mesh=mesh,
        compiler_params=pltpu.CompilerParams(needs_layout_passes=False),
        scratch_types=[
            pltpu.VMEM((SLAB, CHUNK), jnp.int32),
            pltpu.VMEM((SLAB, CHUNK), jnp.int32),
            pltpu.VMEM((CHUNK, HID), jnp.float32),
            pltpu.VMEM((CHUNK, HID), jnp.float32),
            pltpu.SemaphoreType.DMA,
            pltpu.SemaphoreType.DMA,
            pltpu.SemaphoreType.DMA,
            pltpu.SemaphoreType.DMA,
            pltpu.VMEM_SHARED((NP, HID), jnp.float32),
        ],
    )(y, row_p, col_p, zeros_rpt)


def _tc_final_body(aggs_ref, y_ref, dinv_ref, b1_ref, batch_ref, w2_ref,
                   b2_ref, out_ref, sums_ref, counts_ref):
    i = pl.program_id(0)

    @pl.when(i == 0)
    def _():
        sums_ref[...] = jnp.zeros_like(sums_ref)
        counts_ref[...] = jnp.zeros_like(counts_ref)

    h = jax.nn.relu(
        dinv_ref[...] * (aggs_ref[...] + y_ref[...]) + b1_ref[...])
    b = batch_ref[0, 0, :]
    gid = lax.broadcasted_iota(jnp.int32, (G, BLK), 0)
    oh = (b[None, :] == gid).astype(jnp.float32)
    sums_ref[...] += jnp.dot(oh, h, preferred_element_type=jnp.float32)
    counts_ref[...] += jnp.broadcast_to(
        jnp.sum(oh, axis=1, keepdims=True), (G, HID))

    @pl.when(i == NBLK - 1)
    def _():
        pooled = sums_ref[...] / jnp.maximum(counts_ref[...], 1.0)
        logits = (jnp.dot(pooled, w2_ref[...],
                          preferred_element_type=jnp.float32) + b2_ref[...])
        lane = lax.broadcasted_iota(jnp.int32, (G, HID), 1)
        valid = lane < OUT
        neg = jnp.full_like(logits, -jnp.inf)
        m = jnp.max(jnp.where(valid, logits, neg), axis=1, keepdims=True)
        e = jnp.where(valid, jnp.exp(logits - m), 0.0)
        lse = jnp.log(jnp.sum(e, axis=1, keepdims=True))
        out_ref[...] = logits - m - lse


def _tc_final(aggs, y, dinv, b1, batch_p, W2p, b2p):
    return pl.pallas_call(
        _tc_final_body,
        grid=(NBLK,),
        in_specs=[
            pl.BlockSpec((BLK, HID), lambda i: (i, 0)),
            pl.BlockSpec((BLK, HID), lambda i: (i, 0)),
            pl.BlockSpec((BLK, 1), lambda i: (i, 0)),
            pl.BlockSpec((1, HID), lambda i: (0, 0)),
            pl.BlockSpec((1, 1, BLK), lambda i: (i, 0, 0)),
            pl.BlockSpec((HID, HID), lambda i: (0, 0)),
            pl.BlockSpec((1, HID), lambda i: (0, 0)),
        ],
        out_specs=pl.BlockSpec((G, HID), lambda i: (0, 0)),
        out_shape=jax.ShapeDtypeStruct((G, HID), jnp.float32),
        scratch_shapes=[
            pltpu.VMEM((G, HID), jnp.float32),
            pltpu.VMEM((G, HID), jnp.float32),
        ],
    )(aggs, y, dinv, b1, batch_p, W2p, b2p)


def kernel(x, edge_index, batch, W1, b1, W2, b2):
    pad = E_PAD - E
    row = jnp.concatenate([edge_index[0], jnp.zeros((pad,), jnp.int32)])
    dummy = N + (jnp.arange(pad, dtype=jnp.int32) % (NP - N))
    col = jnp.concatenate([edge_index[1], dummy])
    row_p = row.reshape(TOT_CHUNKS, CHUNK)
    col_p = col.reshape(TOT_CHUNKS, CHUNK)
    x_p = jnp.pad(x, ((0, NP - N), (0, 0)))
    batch_p = jnp.concatenate(
        [batch, jnp.full((NP - N,), G, jnp.int32)]).reshape(NBLK, 1, BLK)
    zeros_rpt = jnp.zeros((RPT, HID), jnp.float32)
    W2p = jnp.pad(W2, ((0, 0), (0, HID - OUT)))
    b2p = jnp.pad(b2, (0, HID - OUT)).reshape(1, HID)
    b1r = b1.reshape(1, HID)

    parts = _sc_degree(col.reshape(NW, EPW))
    y, dinv = _tc_scale(x_p, W1, parts)
    aggs = _sc_scatter(y, row_p, col_p, zeros_rpt)
    outp = _tc_final(aggs, y, dinv, b1r, batch_p, W2p, b2p)
    return outp[:, :OUT]


# speedup vs baseline: 2.8807x; 2.8807x over previous
import functools

import jax
import jax.numpy as jnp
from jax import lax
from jax.experimental import pallas as pl
from jax.experimental.pallas import tpu as pltpu
from jax.experimental.pallas import tpu_sc as plsc

N = 10000
F = 128
HID = 128
OUT = 3
G = 64
E = 320000

NC = 2
NS = 16
NW = NC * NS
L = 16

CHUNK = 64
NP = 10240
EPW = NP
E_PAD = NW * EPW
TOT_CHUNKS = E_PAD // CHUNK
CPT = TOT_CHUNKS // NW
SLAB = 40
RPT = NP // NS

BLK = 2048
NBLK = NP // BLK


DCH = 512


def _sc_deg_body(col_hbm, parts_hbm, col_v, deg_v):
    c = lax.axis_index("c")
    s = lax.axis_index("s")
    wid = s * NC + c

    def zero(i, _):
        deg_v[pl.ds(i * L, L)] = jnp.zeros((L,), jnp.float32)
        return 0

    lax.fori_loop(0, NP // L, zero, 0)

    ones = jnp.ones((L,), jnp.float32)

    def outer(t, _):
        pltpu.sync_copy(col_hbm.at[wid, pl.ds(t * DCH, DCH)], col_v)

        def body(k, _):
            idx = col_v[pl.ds(k * L, L)]
            plsc.addupdate_scatter(deg_v, [idx], ones)
            return 0

        lax.fori_loop(0, DCH // L, body, 0)
        return 0

    lax.fori_loop(0, EPW // DCH, outer, 0)
    pltpu.sync_copy(deg_v, parts_hbm.at[wid])


def _sc_degree(col_flat):
    mesh = plsc.VectorSubcoreMesh(core_axis_name="c", subcore_axis_name="s")
    return pl.kernel(
        _sc_deg_body,
        out_type=jax.ShapeDtypeStruct((NW, NP), jnp.float32),
        mesh=mesh,
        compiler_params=pltpu.CompilerParams(needs_layout_passes=False),
        scratch_types=[
            pltpu.VMEM((DCH,), jnp.int32),
            pltpu.VMEM((NP,), jnp.float32),
        ],
    )(col_flat)


def _tc_scale_body(x_ref, w_ref, parts_ref, y_ref, dinv_ref):
    deg = jnp.sum(parts_ref[...], axis=0) + 1.0
    dinv = lax.rsqrt(deg).reshape(BLK, 1)
    xw = jnp.dot(x_ref[...], w_ref[...], preferred_element_type=jnp.float32)
    y_ref[...] = xw * dinv
    dinv_ref[...] = dinv


def _tc_scale(x_p, W1, parts):
    return pl.pallas_call(
        _tc_scale_body,
        grid=(NBLK,),
        in_specs=[
            pl.BlockSpec((BLK, F), lambda i: (i, 0)),
            pl.BlockSpec((F, HID), lambda i: (0, 0)),
            pl.BlockSpec((NW, BLK), lambda i: (0, i)),
        ],
        out_specs=[
            pl.BlockSpec((BLK, HID), lambda i: (i, 0)),
            pl.BlockSpec((BLK, 1), lambda i: (i, 0)),
        ],
        out_shape=[
            jax.ShapeDtypeStruct((NP, HID), jnp.float32),
            jax.ShapeDtypeStruct((NP, 1), jnp.float32),
        ],
    )(x_p, W1, parts)


def _sc_scatter_body(y_hbm, row_hbm, col_hbm, z_hbm, out_hbm,
                     row_v, col_v, buf0, buf1, gsem0, gsem1, ssem0, ssem1,
                     acc):
    c = lax.axis_index("c")
    s = lax.axis_index("s")
    base = (c * NS + s) * CPT
    pltpu.sync_copy(z_hbm, acc.at[pl.ds(s * RPT, RPT)])
    plsc.subcore_barrier()

    pairs = SLAB // 2

    def body(j, _):
        j0 = 2 * j
        j1 = 2 * j + 1
        pltpu.make_async_copy(y_hbm.at[row_v.at[j0]], buf0, gsem0).wait()
        pltpu.async_copy(buf0, acc.at[col_v.at[j0]], ssem0, add=True)
        pltpu.make_async_copy(y_hbm.at[row_v.at[j1]], buf1, gsem1).wait()
        pltpu.async_copy(buf1, acc.at[col_v.at[j1]], ssem1, add=True)

        @pl.when(j < pairs - 1)
        def _():
            pltpu.make_async_copy(buf0, acc.at[col_v.at[j0]], ssem0).wait()
            pltpu.async_copy(y_hbm.at[row_v.at[j0 + 2]], buf0, gsem0)
            pltpu.make_async_copy(buf1, acc.at[col_v.at[j1]], ssem1).wait()
            pltpu.async_copy(y_hbm.at[row_v.at[j1 + 2]], buf1, gsem1)

        return 0

    def slab_body(t, _):
        off = base + t * SLAB
        pltpu.sync_copy(row_hbm.at[pl.ds(off, SLAB)], row_v)
        pltpu.sync_copy(col_hbm.at[pl.ds(off, SLAB)], col_v)
        pltpu.async_copy(y_hbm.at[row_v.at[0]], buf0, gsem0)
        pltpu.async_copy(y_hbm.at[row_v.at[1]], buf1, gsem1)
        lax.fori_loop(0, pairs, body, 0)
        pltpu.make_async_copy(buf0, acc.at[col_v.at[0]], ssem0).wait()
        pltpu.make_async_copy(buf1, acc.at[col_v.at[1]], ssem1).wait()
        return 0

    lax.fori_loop(0, CPT // SLAB, slab_body, 0)
    plsc.subcore_barrier()
    pltpu.sync_copy(acc.at[pl.ds(s * RPT, RPT)],
                    out_hbm.at[c, pl.ds(s * RPT, RPT)])


def _sc_scatter(y, row_p, col_p, zeros_rpt):
    mesh = plsc.VectorSubcoreMesh(core_axis_name="c", subcore_axis_name="s")
    return pl.kernel(
        _sc_scatter_body,
        out_type=jax.ShapeDtypeStruct((NC, NP, HID), jnp.float32),
        mesh=mesh,
        compiler_params=pltpu.CompilerParams(needs_layout_passes=False),
        scratch_types=[
            pltpu.VMEM((SLAB, CHUNK), jnp.int32),
            pltpu.VMEM((SLAB, CHUNK), jnp.int32),
            pltpu.VMEM((CHUNK, HID), jnp.float32),
            pltpu.VMEM((CHUNK, HID), jnp.float32),
            pltpu.SemaphoreType.DMA,
            pltpu.SemaphoreType.DMA,
            pltpu.SemaphoreType.DMA,
            pltpu.SemaphoreType.DMA,
            pltpu.VMEM_SHARED((NP, HID), jnp.float32),
        ],
    )(y, row_p, col_p, zeros_rpt)


def _tc_final_body(aggs_ref, y_ref, dinv_ref, b1_ref, batch_ref, w2_ref,
                   b2_ref, out_ref, sums_ref, counts_ref):
    i = pl.program_id(0)

    @pl.when(i == 0)
    def _():
        sums_ref[...] = jnp.zeros_like(sums_ref)
        counts_ref[...] = jnp.zeros_like(counts_ref)

    agg = aggs_ref[0] + aggs_ref[1]
    h = jax.nn.relu(dinv_ref[...] * (agg + y_ref[...]) + b1_ref[...])
    b = batch_ref[0, 0, :]
    gid = lax.broadcasted_iota(jnp.int32, (G, BLK), 0)
    oh = (b[None, :] == gid).astype(jnp.float32)
    sums_ref[...] += jnp.dot(oh, h, preferred_element_type=jnp.float32)
    counts_ref[...] += jnp.broadcast_to(
        jnp.sum(oh, axis=1, keepdims=True), (G, HID))

    @pl.when(i == NBLK - 1)
    def _():
        pooled = sums_ref[...] / jnp.maximum(counts_ref[...], 1.0)
        logits = (jnp.dot(pooled, w2_ref[...],
                          preferred_element_type=jnp.float32) + b2_ref[...])
        lane = lax.broadcasted_iota(jnp.int32, (G, HID), 1)
        valid = lane < OUT
        neg = jnp.full_like(logits, -jnp.inf)
        m = jnp.max(jnp.where(valid, logits, neg), axis=1, keepdims=True)
        e = jnp.where(valid, jnp.exp(logits - m), 0.0)
        lse = jnp.log(jnp.sum(e, axis=1, keepdims=True))
        out_ref[...] = logits - m - lse


def _tc_final(aggs, y, dinv, b1, batch_p, W2p, b2p):
    return pl.pallas_call(
        _tc_final_body,
        grid=(NBLK,),
        in_specs=[
            pl.BlockSpec((NC, BLK, HID), lambda i: (0, i, 0)),
            pl.BlockSpec((BLK, HID), lambda i: (i, 0)),
            pl.BlockSpec((BLK, 1), lambda i: (i, 0)),
            pl.BlockSpec((1, HID), lambda i: (0, 0)),
            pl.BlockSpec((1, 1, BLK), lambda i: (i, 0, 0)),
            pl.BlockSpec((HID, HID), lambda i: (0, 0)),
            pl.BlockSpec((1, HID), lambda i: (0, 0)),
        ],
        out_specs=pl.BlockSpec((G, HID), lambda i: (0, 0)),
        out_shape=jax.ShapeDtypeStruct((G, HID), jnp.float32),
        scratch_shapes=[
            pltpu.VMEM((G, HID), jnp.float32),
            pltpu.VMEM((G, HID), jnp.float32),
        ],
    )(aggs, y, dinv, b1, batch_p, W2p, b2p)


def kernel(x, edge_index, batch, W1, b1, W2, b2):
    pad = E_PAD - E
    idx = jnp.arange(pad, dtype=jnp.int32)
    row = jnp.concatenate([edge_index[0], idx % N])
    col = jnp.concatenate([edge_index[1], N + idx % (NP - N)])
    row_p = row.reshape(TOT_CHUNKS, CHUNK)
    col_p = col.reshape(TOT_CHUNKS, CHUNK)
    x_p = jnp.pad(x, ((0, NP - N), (0, 0)))
    batch_p = jnp.concatenate(
        [batch, jnp.full((NP - N,), G, jnp.int32)]).reshape(NBLK, 1, BLK)
    zeros_rpt = jnp.zeros((RPT, HID), jnp.float32)
    W2p = jnp.pad(W2, ((0, 0), (0, HID - OUT)))
    b2p = jnp.pad(b2, (0, HID - OUT)).reshape(1, HID)
    b1r = b1.reshape(1, HID)

    parts = _sc_degree(col.reshape(NW, EPW))
    y, dinv = _tc_scale(x_p, W1, parts)
    aggs = _sc_scatter(y, row_p, col_p, zeros_rpt)
    outp = _tc_final(aggs, y, dinv, b1r, batch_p, W2p, b2p)
    return outp[:, :OUT]


# speedup vs baseline: 3.0722x; 1.0665x over previous
import functools

import jax
import jax.numpy as jnp
from jax import lax
from jax.experimental import pallas as pl
from jax.experimental.pallas import tpu as pltpu
from jax.experimental.pallas import tpu_sc as plsc

N = 10000
F = 128
HID = 128
OUT = 3
G = 64
E = 320000

NC = 2
NS = 16
NW = NC * NS
L = 16

CHUNK = 128
NP = 10240
EPW = NP
E_PAD = NW * EPW
TOT_CHUNKS = E_PAD // CHUNK
CPT = TOT_CHUNKS // NW
SLAB = 16
RPT = NP // NS

BLK = 2048
NBLK = NP // BLK


DCH = 512


def _sc_deg_body(col_hbm, parts_hbm, col_v, deg_v):
    c = lax.axis_index("c")
    s = lax.axis_index("s")
    wid = s * NC + c

    def zero(i, _):
        deg_v[pl.ds(i * L, L)] = jnp.zeros((L,), jnp.float32)
        return 0

    lax.fori_loop(0, NP // L, zero, 0)

    ones = jnp.ones((L,), jnp.float32)

    def outer(t, _):
        pltpu.sync_copy(col_hbm.at[wid, pl.ds(t * DCH, DCH)], col_v)

        def body(k, _):
            idx = col_v[pl.ds(k * L, L)]
            plsc.addupdate_scatter(deg_v, [idx], ones)
            return 0

        lax.fori_loop(0, DCH // L, body, 0)
        return 0

    lax.fori_loop(0, EPW // DCH, outer, 0)
    pltpu.sync_copy(deg_v, parts_hbm.at[wid])


def _sc_degree(col_flat):
    mesh = plsc.VectorSubcoreMesh(core_axis_name="c", subcore_axis_name="s")
    return pl.kernel(
        _sc_deg_body,
        out_type=jax.ShapeDtypeStruct((NW, NP), jnp.float32),
        mesh=mesh,
        compiler_params=pltpu.CompilerParams(needs_layout_passes=False),
        scratch_types=[
            pltpu.VMEM((DCH,), jnp.int32),
            pltpu.VMEM((NP,), jnp.float32),
        ],
    )(col_flat)


def _tc_scale_body(x_ref, w_ref, parts_ref, y_ref, dinv_ref):
    deg = jnp.sum(parts_ref[...], axis=0) + 1.0
    dinv = lax.rsqrt(deg).reshape(BLK, 1)
    xw = jnp.dot(x_ref[...], w_ref[...], preferred_element_type=jnp.float32)
    y_ref[...] = xw * dinv
    dinv_ref[...] = dinv


def _tc_scale(x_p, W1, parts):
    return pl.pallas_call(
        _tc_scale_body,
        grid=(NBLK,),
        in_specs=[
            pl.BlockSpec((BLK, F), lambda i: (i, 0)),
            pl.BlockSpec((F, HID), lambda i: (0, 0)),
            pl.BlockSpec((NW, BLK), lambda i: (0, i)),
        ],
        out_specs=[
            pl.BlockSpec((BLK, HID), lambda i: (i, 0)),
            pl.BlockSpec((BLK, 1), lambda i: (i, 0)),
        ],
        out_shape=[
            jax.ShapeDtypeStruct((NP, HID), jnp.float32),
            jax.ShapeDtypeStruct((NP, 1), jnp.float32),
        ],
    )(x_p, W1, parts)


def _sc_scatter_body(y_hbm, row_hbm, col_hbm, z_hbm, out_hbm,
                     row_v, col_v, buf0, buf1, gsem0, gsem1, ssem0, ssem1,
                     acc):
    c = lax.axis_index("c")
    s = lax.axis_index("s")
    base = (c * NS + s) * CPT
    pltpu.sync_copy(z_hbm, acc.at[pl.ds(s * RPT, RPT)])
    plsc.subcore_barrier()

    pairs = SLAB // 2

    def body(j, _):
        j0 = 2 * j
        j1 = 2 * j + 1
        pltpu.make_async_copy(y_hbm.at[row_v.at[j0]], buf0, gsem0).wait()
        pltpu.async_copy(buf0, acc.at[col_v.at[j0]], ssem0, add=True)
        pltpu.make_async_copy(y_hbm.at[row_v.at[j1]], buf1, gsem1).wait()
        pltpu.async_copy(buf1, acc.at[col_v.at[j1]], ssem1, add=True)

        @pl.when(j < pairs - 1)
        def _():
            pltpu.make_async_copy(buf0, acc.at[col_v.at[j0]], ssem0).wait()
            pltpu.async_copy(y_hbm.at[row_v.at[j0 + 2]], buf0, gsem0)
            pltpu.make_async_copy(buf1, acc.at[col_v.at[j1]], ssem1).wait()
            pltpu.async_copy(y_hbm.at[row_v.at[j1 + 2]], buf1, gsem1)

        return 0

    def slab_body(t, _):
        off = base + t * SLAB
        pltpu.sync_copy(row_hbm.at[pl.ds(off, SLAB)], row_v)
        pltpu.sync_copy(col_hbm.at[pl.ds(off, SLAB)], col_v)
        pltpu.async_copy(y_hbm.at[row_v.at[0]], buf0, gsem0)
        pltpu.async_copy(y_hbm.at[row_v.at[1]], buf1, gsem1)
        lax.fori_loop(0, pairs, body, 0)
        pltpu.make_async_copy(buf0, acc.at[col_v.at[0]], ssem0).wait()
        pltpu.make_async_copy(buf1, acc.at[col_v.at[1]], ssem1).wait()
        return 0

    lax.fori_loop(0, CPT // SLAB, slab_body, 0)
    plsc.subcore_barrier()
    pltpu.sync_copy(acc.at[pl.ds(s * RPT, RPT)],
                    out_hbm.at[c, pl.ds(s * RPT, RPT)])


def _sc_scatter(y, row_p, col_p, zeros_rpt):
    mesh = plsc.VectorSubcoreMesh(core_axis_name="c", subcore_axis_name="s")
    return pl.kernel(
        _sc_scatter_body,
        out_type=jax.ShapeDtypeStruct((NC, NP, HID), jnp.float32),
        mesh=mesh,
        compiler_params=pltpu.CompilerParams(needs_layout_passes=False),
        scratch_types=[
            pltpu.VMEM((SLAB, CHUNK), jnp.int32),
            pltpu.VMEM((SLAB, CHUNK), jnp.int32),
            pltpu.VMEM((CHUNK, HID), jnp.float32),
            pltpu.VMEM((CHUNK, HID), jnp.float32),
            pltpu.SemaphoreType.DMA,
            pltpu.SemaphoreType.DMA,
            pltpu.SemaphoreType.DMA,
            pltpu.SemaphoreType.DMA,
            pltpu.VMEM_SHARED((NP, HID), jnp.float32),
        ],
    )(y, row_p, col_p, zeros_rpt)


def _tc_final_body(aggs_ref, y_ref, dinv_ref, b1_ref, batch_ref, w2_ref,
                   b2_ref, out_ref, sums_ref, counts_ref):
    i = pl.program_id(0)

    @pl.when(i == 0)
    def _():
        sums_ref[...] = jnp.zeros_like(sums_ref)
        counts_ref[...] = jnp.zeros_like(counts_ref)

    agg = aggs_ref[0] + aggs_ref[1]
    h = jax.nn.relu(dinv_ref[...] * (agg + y_ref[...]) + b1_ref[...])
    b = batch_ref[0, 0, :]
    gid = lax.broadcasted_iota(jnp.int32, (G, BLK), 0)
    oh = (b[None, :] == gid).astype(jnp.float32)
    sums_ref[...] += jnp.dot(oh, h, preferred_element_type=jnp.float32)
    counts_ref[...] += jnp.broadcast_to(
        jnp.sum(oh, axis=1, keepdims=True), (G, HID))

    @pl.when(i == NBLK - 1)
    def _():
        pooled = sums_ref[...] / jnp.maximum(counts_ref[...], 1.0)
        logits = (jnp.dot(pooled, w2_ref[...],
                          preferred_element_type=jnp.float32) + b2_ref[...])
        lane = lax.broadcasted_iota(jnp.int32, (G, HID), 1)
        valid = lane < OUT
        neg = jnp.full_like(logits, -jnp.inf)
        m = jnp.max(jnp.where(valid, logits, neg), axis=1, keepdims=True)
        e = jnp.where(valid, jnp.exp(logits - m), 0.0)
        lse = jnp.log(jnp.sum(e, axis=1, keepdims=True))
        out_ref[...] = logits - m - lse


def _tc_final(aggs, y, dinv, b1, batch_p, W2p, b2p):
    return pl.pallas_call(
        _tc_final_body,
        grid=(NBLK,),
        in_specs=[
            pl.BlockSpec((NC, BLK, HID), lambda i: (0, i, 0)),
            pl.BlockSpec((BLK, HID), lambda i: (i, 0)),
            pl.BlockSpec((BLK, 1), lambda i: (i, 0)),
            pl.BlockSpec((1, HID), lambda i: (0, 0)),
            pl.BlockSpec((1, 1, BLK), lambda i: (i, 0, 0)),
            pl.BlockSpec((HID, HID), lambda i: (0, 0)),
            pl.BlockSpec((1, HID), lambda i: (0, 0)),
        ],
        out_specs=pl.BlockSpec((G, HID), lambda i: (0, 0)),
        out_shape=jax.ShapeDtypeStruct((G, HID), jnp.float32),
        scratch_shapes=[
            pltpu.VMEM((G, HID), jnp.float32),
            pltpu.VMEM((G, HID), jnp.float32),
        ],
    )(aggs, y, dinv, b1, batch_p, W2p, b2p)


def kernel(x, edge_index, batch, W1, b1, W2, b2):
    pad = E_PAD - E
    idx = jnp.arange(pad, dtype=jnp.int32)
    row = jnp.concatenate([edge_index[0], idx % N])
    col = jnp.concatenate([edge_index[1], N + idx % (NP - N)])
    row_p = row.reshape(TOT_CHUNKS, CHUNK)
    col_p = col.reshape(TOT_CHUNKS, CHUNK)
    x_p = jnp.pad(x, ((0, NP - N), (0, 0)))
    batch_p = jnp.concatenate(
        [batch, jnp.full((NP - N,), G, jnp.int32)]).reshape(NBLK, 1, BLK)
    zeros_rpt = jnp.zeros((RPT, HID), jnp.float32)
    W2p = jnp.pad(W2, ((0, 0), (0, HID - OUT)))
    b2p = jnp.pad(b2, (0, HID - OUT)).reshape(1, HID)
    b1r = b1.reshape(1, HID)

    parts = _sc_degree(col.reshape(NW, EPW))
    y, dinv = _tc_scale(x_p, W1, parts)
    aggs = _sc_scatter(y, row_p, col_p, zeros_rpt)
    outp = _tc_final(aggs, y, dinv, b1r, batch_p, W2p, b2p)
    return outp[:, :OUT]


# speedup vs baseline: 3.2142x; 1.0462x over previous
import functools

import jax
import jax.numpy as jnp
from jax import lax
from jax.experimental import pallas as pl
from jax.experimental.pallas import tpu as pltpu
from jax.experimental.pallas import tpu_sc as plsc

N = 10000
F = 128
HID = 128
OUT = 3
G = 64
E = 320000

NC = 2
NS = 16
NW = NC * NS
L = 16

CHUNK = 128
NP = 10240
EPW = NP
E_PAD = NW * EPW
TOT_CHUNKS = E_PAD // CHUNK
CPT = TOT_CHUNKS // NW
SLAB = 16
RPT = NP // NS

BLK = 2048
NBLK = NP // BLK


EPW_D = E // NW
DCH = 400


def _sc_deg_body(ei_hbm, parts_hbm, col_v, deg_v):
    c = lax.axis_index("c")
    s = lax.axis_index("s")
    wid = s * NC + c

    def zero(i, _):
        deg_v[pl.ds(i * L, L)] = jnp.zeros((L,), jnp.float32)
        return 0

    lax.fori_loop(0, NP // L, zero, 0)

    ones = jnp.ones((L,), jnp.float32)

    def outer(t, _):
        pltpu.sync_copy(ei_hbm.at[pl.ds(E + wid * EPW_D + t * DCH, DCH)],
                        col_v)

        def body(k, _):
            idx = col_v[pl.ds(k * L, L)]
            plsc.addupdate_scatter(deg_v, [idx], ones)
            return 0

        lax.fori_loop(0, DCH // L, body, 0)
        return 0

    lax.fori_loop(0, EPW_D // DCH, outer, 0)
    pltpu.sync_copy(deg_v, parts_hbm.at[wid])


def _sc_degree(edge_index):
    mesh = plsc.VectorSubcoreMesh(core_axis_name="c", subcore_axis_name="s")
    return pl.kernel(
        _sc_deg_body,
        out_type=jax.ShapeDtypeStruct((NW, NP), jnp.float32),
        mesh=mesh,
        compiler_params=pltpu.CompilerParams(needs_layout_passes=False),
        scratch_types=[
            pltpu.VMEM((DCH,), jnp.int32),
            pltpu.VMEM((NP,), jnp.float32),
        ],
    )(edge_index)


def _tc_scale_body(x_ref, w_ref, parts_ref, y_ref, dinv_ref):
    deg = jnp.sum(parts_ref[...], axis=0) + 1.0
    dinv = lax.rsqrt(deg).reshape(BLK, 1)
    xw = jnp.dot(x_ref[...], w_ref[...], preferred_element_type=jnp.float32)
    y_ref[...] = xw * dinv
    dinv_ref[...] = dinv


def _tc_scale(x_p, W1, parts):
    return pl.pallas_call(
        _tc_scale_body,
        grid=(NBLK,),
        in_specs=[
            pl.BlockSpec((BLK, F), lambda i: (i, 0)),
            pl.BlockSpec((F, HID), lambda i: (0, 0)),
            pl.BlockSpec((NW, BLK), lambda i: (0, i)),
        ],
        out_specs=[
            pl.BlockSpec((BLK, HID), lambda i: (i, 0)),
            pl.BlockSpec((BLK, 1), lambda i: (i, 0)),
        ],
        out_shape=[
            jax.ShapeDtypeStruct((NP, HID), jnp.float32),
            jax.ShapeDtypeStruct((NP, 1), jnp.float32),
        ],
    )(x_p, W1, parts)


def _sc_scatter_body(y_hbm, row_hbm, col_hbm, z_hbm, out_hbm,
                     row_v, col_v, buf0, buf1, gsem0, gsem1, ssem0, ssem1,
                     acc):
    c = lax.axis_index("c")
    s = lax.axis_index("s")
    base = (c * NS + s) * CPT
    pltpu.sync_copy(z_hbm, acc.at[pl.ds(s * RPT, RPT)])
    plsc.subcore_barrier()

    pairs = SLAB // 2

    def body(j, _):
        j0 = 2 * j
        j1 = 2 * j + 1
        pltpu.make_async_copy(y_hbm.at[row_v.at[j0]], buf0, gsem0).wait()
        pltpu.async_copy(buf0, acc.at[col_v.at[j0]], ssem0, add=True)
        pltpu.make_async_copy(y_hbm.at[row_v.at[j1]], buf1, gsem1).wait()
        pltpu.async_copy(buf1, acc.at[col_v.at[j1]], ssem1, add=True)

        @pl.when(j < pairs - 1)
        def _():
            pltpu.make_async_copy(buf0, acc.at[col_v.at[j0]], ssem0).wait()
            pltpu.async_copy(y_hbm.at[row_v.at[j0 + 2]], buf0, gsem0)
            pltpu.make_async_copy(buf1, acc.at[col_v.at[j1]], ssem1).wait()
            pltpu.async_copy(y_hbm.at[row_v.at[j1 + 2]], buf1, gsem1)

        return 0

    def slab_body(t, _):
        off = base + t * SLAB
        pltpu.sync_copy(row_hbm.at[pl.ds(off, SLAB)], row_v)
        pltpu.sync_copy(col_hbm.at[pl.ds(off, SLAB)], col_v)
        pltpu.async_copy(y_hbm.at[row_v.at[0]], buf0, gsem0)
        pltpu.async_copy(y_hbm.at[row_v.at[1]], buf1, gsem1)
        lax.fori_loop(0, pairs, body, 0)
        pltpu.make_async_copy(buf0, acc.at[col_v.at[0]], ssem0).wait()
        pltpu.make_async_copy(buf1, acc.at[col_v.at[1]], ssem1).wait()
        return 0

    lax.fori_loop(0, CPT // SLAB, slab_body, 0)
    plsc.subcore_barrier()
    pltpu.sync_copy(acc.at[pl.ds(s * RPT, RPT)],
                    out_hbm.at[c, pl.ds(s * RPT, RPT)])


def _sc_scatter(y, row_p, col_p, zeros_rpt):
    mesh = plsc.VectorSubcoreMesh(core_axis_name="c", subcore_axis_name="s")
    return pl.kernel(
        _sc_scatter_body,
        out_type=jax.ShapeDtypeStruct((NC, NP, HID), jnp.float32),
        mesh=mesh,
        compiler_params=pltpu.CompilerParams(needs_layout_passes=False),
        scratch_types=[
            pltpu.VMEM((SLAB, CHUNK), jnp.int32),
            pltpu.VMEM((SLAB, CHUNK), jnp.int32),
            pltpu.VMEM((CHUNK, HID), jnp.float32),
            pltpu.VMEM((CHUNK, HID), jnp.float32),
            pltpu.SemaphoreType.DMA,
            pltpu.SemaphoreType.DMA,
            pltpu.SemaphoreType.DMA,
            pltpu.SemaphoreType.DMA,
            pltpu.VMEM_SHARED((NP, HID), jnp.float32),
        ],
    )(y, row_p, col_p, zeros_rpt)


def _tc_final_body(aggs_ref, y_ref, dinv_ref, b1_ref, batch_ref, w2_ref,
                   b2_ref, out_ref, sums_ref, counts_ref):
    i = pl.program_id(0)

    @pl.when(i == 0)
    def _():
        sums_ref[...] = jnp.zeros_like(sums_ref)
        counts_ref[...] = jnp.zeros_like(counts_ref)

    agg = aggs_ref[0] + aggs_ref[1]
    h = jax.nn.relu(dinv_ref[...] * (agg + y_ref[...]) + b1_ref[...])
    b = batch_ref[0, 0, :]
    gid = lax.broadcasted_iota(jnp.int32, (G, BLK), 0)
    oh = (b[None, :] == gid).astype(jnp.float32)
    sums_ref[...] += jnp.dot(oh, h, preferred_element_type=jnp.float32)
    counts_ref[...] += jnp.broadcast_to(
        jnp.sum(oh, axis=1, keepdims=True), (G, HID))

    @pl.when(i == NBLK - 1)
    def _():
        pooled = sums_ref[...] / jnp.maximum(counts_ref[...], 1.0)
        logits = (jnp.dot(pooled, w2_ref[...],
                          preferred_element_type=jnp.float32) + b2_ref[...])
        lane = lax.broadcasted_iota(jnp.int32, (G, HID), 1)
        valid = lane < OUT
        neg = jnp.full_like(logits, -jnp.inf)
        m = jnp.max(jnp.where(valid, logits, neg), axis=1, keepdims=True)
        e = jnp.where(valid, jnp.exp(logits - m), 0.0)
        lse = jnp.log(jnp.sum(e, axis=1, keepdims=True))
        out_ref[...] = logits - m - lse


def _tc_final(aggs, y, dinv, b1, batch_p, W2p, b2p):
    return pl.pallas_call(
        _tc_final_body,
        grid=(NBLK,),
        in_specs=[
            pl.BlockSpec((NC, BLK, HID), lambda i: (0, i, 0)),
            pl.BlockSpec((BLK, HID), lambda i: (i, 0)),
            pl.BlockSpec((BLK, 1), lambda i: (i, 0)),
            pl.BlockSpec((1, HID), lambda i: (0, 0)),
            pl.BlockSpec((1, 1, BLK), lambda i: (i, 0, 0)),
            pl.BlockSpec((HID, HID), lambda i: (0, 0)),
            pl.BlockSpec((1, HID), lambda i: (0, 0)),
        ],
        out_specs=pl.BlockSpec((G, HID), lambda i: (0, 0)),
        out_shape=jax.ShapeDtypeStruct((G, HID), jnp.float32),
        scratch_shapes=[
            pltpu.VMEM((G, HID), jnp.float32),
            pltpu.VMEM((G, HID), jnp.float32),
        ],
    )(aggs, y, dinv, b1, batch_p, W2p, b2p)


def kernel(x, edge_index, batch, W1, b1, W2, b2):
    pad = E_PAD - E
    idx = jnp.arange(pad, dtype=jnp.int32)
    row = jnp.concatenate([edge_index[0], idx % N])
    col = jnp.concatenate([edge_index[1], N + idx % (NP - N)])
    row_p = row.reshape(TOT_CHUNKS, CHUNK)
    col_p = col.reshape(TOT_CHUNKS, CHUNK)
    x_p = jnp.pad(x, ((0, NP - N), (0, 0)))
    batch_p = jnp.concatenate(
        [batch, jnp.full((NP - N,), G, jnp.int32)]).reshape(NBLK, 1, BLK)
    zeros_rpt = jnp.zeros((RPT, HID), jnp.float32)
    W2p = jnp.pad(W2, ((0, 0), (0, HID - OUT)))
    b2p = jnp.pad(b2, (0, HID - OUT)).reshape(1, HID)
    b1r = b1.reshape(1, HID)

    parts = _sc_degree(edge_index.reshape(-1))
    y, dinv = _tc_scale(x_p, W1, parts)
    aggs = _sc_scatter(y, row_p, col_p, zeros_rpt)
    outp = _tc_final(aggs, y, dinv, b1r, batch_p, W2p, b2p)
    return outp[:, :OUT]


# speedup vs baseline: 3.4736x; 1.0807x over previous
import functools

import jax
import jax.numpy as jnp
from jax import lax
from jax.experimental import pallas as pl
from jax.experimental.pallas import tpu as pltpu
from jax.experimental.pallas import tpu_sc as plsc

N = 10000
F = 128
HID = 128
OUT = 3
G = 64
E = 320000

NC = 2
NS = 16
NW = NC * NS
L = 16

CHUNK = 64
NBUF = 4
NP = 10240
EPW = NP
E_PAD = NW * EPW
TOT_CHUNKS = E_PAD // CHUNK
CPT = TOT_CHUNKS // NW
SLAB = 16
RPT = NP // NS

BLK = 2048
NBLK = NP // BLK


EPW_D = E // NW
DCH = 400


def _sc_deg_body(ei_hbm, parts_hbm, col_v, deg_v):
    c = lax.axis_index("c")
    s = lax.axis_index("s")
    wid = s * NC + c

    def zero(i, _):
        deg_v[pl.ds(i * L, L)] = jnp.zeros((L,), jnp.float32)
        return 0

    lax.fori_loop(0, NP // L, zero, 0)

    ones = jnp.ones((L,), jnp.float32)

    def outer(t, _):
        pltpu.sync_copy(ei_hbm.at[pl.ds(E + wid * EPW_D + t * DCH, DCH)],
                        col_v)

        def body(k, _):
            idx = col_v[pl.ds(k * L, L)]
            plsc.addupdate_scatter(deg_v, [idx], ones)
            return 0

        lax.fori_loop(0, DCH // L, body, 0)
        return 0

    lax.fori_loop(0, EPW_D // DCH, outer, 0)
    pltpu.sync_copy(deg_v, parts_hbm.at[wid])


def _sc_degree(edge_index):
    mesh = plsc.VectorSubcoreMesh(core_axis_name="c", subcore_axis_name="s")
    return pl.kernel(
        _sc_deg_body,
        out_type=jax.ShapeDtypeStruct((NW, NP), jnp.float32),
        mesh=mesh,
        compiler_params=pltpu.CompilerParams(needs_layout_passes=False),
        scratch_types=[
            pltpu.VMEM((DCH,), jnp.int32),
            pltpu.VMEM((NP,), jnp.float32),
        ],
    )(edge_index)


def _tc_scale_body(x_ref, w_ref, parts_ref, y_ref, dinv_ref):
    deg = jnp.sum(parts_ref[...], axis=0) + 1.0
    dinv = lax.rsqrt(deg).reshape(BLK, 1)
    xw = jnp.dot(x_ref[...], w_ref[...], preferred_element_type=jnp.float32)
    y_ref[...] = xw * dinv
    dinv_ref[...] = dinv


def _tc_scale(x_p, W1, parts):
    return pl.pallas_call(
        _tc_scale_body,
        grid=(NBLK,),
        in_specs=[
            pl.BlockSpec((BLK, F), lambda i: (i, 0)),
            pl.BlockSpec((F, HID), lambda i: (0, 0)),
            pl.BlockSpec((NW, BLK), lambda i: (0, i)),
        ],
        out_specs=[
            pl.BlockSpec((BLK, HID), lambda i: (i, 0)),
            pl.BlockSpec((BLK, 1), lambda i: (i, 0)),
        ],
        out_shape=[
            jax.ShapeDtypeStruct((NP, HID), jnp.float32),
            jax.ShapeDtypeStruct((NP, 1), jnp.float32),
        ],
    )(x_p, W1, parts)


def _sc_scatter_body(y_hbm, row_hbm, col_hbm, z_hbm, out_hbm,
                     row_v, col_v, buf0, buf1, buf2, buf3,
                     gsem0, gsem1, gsem2, gsem3,
                     ssem0, ssem1, ssem2, ssem3, acc):
    c = lax.axis_index("c")
    s = lax.axis_index("s")
    base = (c * NS + s) * CPT
    bufs = (buf0, buf1, buf2, buf3)
    gsems = (gsem0, gsem1, gsem2, gsem3)
    ssems = (ssem0, ssem1, ssem2, ssem3)
    pltpu.sync_copy(z_hbm, acc.at[pl.ds(s * RPT, RPT)])
    plsc.subcore_barrier()

    groups = SLAB // NBUF

    def body(j, _):
        for b in range(NBUF):
            jb = NBUF * j + b
            pltpu.make_async_copy(y_hbm.at[row_v.at[jb]], bufs[b],
                                  gsems[b]).wait()
            pltpu.async_copy(bufs[b], acc.at[col_v.at[jb]], ssems[b],
                             add=True)

        @pl.when(j < groups - 1)
        def _():
            for b in range(NBUF):
                jb = NBUF * j + b
                pltpu.make_async_copy(bufs[b], acc.at[col_v.at[jb]],
                                      ssems[b]).wait()
                pltpu.async_copy(y_hbm.at[row_v.at[jb + NBUF]], bufs[b],
                                 gsems[b])

        return 0

    def slab_body(t, _):
        off = base + t * SLAB
        pltpu.sync_copy(row_hbm.at[pl.ds(off, SLAB)], row_v)
        pltpu.sync_copy(col_hbm.at[pl.ds(off, SLAB)], col_v)
        for b in range(NBUF):
            pltpu.async_copy(y_hbm.at[row_v.at[b]], bufs[b], gsems[b])
        lax.fori_loop(0, groups, body, 0)
        for b in range(NBUF):
            pltpu.make_async_copy(bufs[b], acc.at[col_v.at[b]],
                                  ssems[b]).wait()
        return 0

    lax.fori_loop(0, CPT // SLAB, slab_body, 0)
    plsc.subcore_barrier()
    pltpu.sync_copy(acc.at[pl.ds(s * RPT, RPT)],
                    out_hbm.at[c, pl.ds(s * RPT, RPT)])


def _sc_scatter(y, row_p, col_p, zeros_rpt):
    mesh = plsc.VectorSubcoreMesh(core_axis_name="c", subcore_axis_name="s")
    return pl.kernel(
        _sc_scatter_body,
        out_type=jax.ShapeDtypeStruct((NC, NP, HID), jnp.float32),
        mesh=mesh,
        compiler_params=pltpu.CompilerParams(needs_layout_passes=False),
        scratch_types=[
            pltpu.VMEM((SLAB, CHUNK), jnp.int32),
            pltpu.VMEM((SLAB, CHUNK), jnp.int32),
            pltpu.VMEM((CHUNK, HID), jnp.float32),
            pltpu.VMEM((CHUNK, HID), jnp.float32),
            pltpu.VMEM((CHUNK, HID), jnp.float32),
            pltpu.VMEM((CHUNK, HID), jnp.float32),
            pltpu.SemaphoreType.DMA,
            pltpu.SemaphoreType.DMA,
            pltpu.SemaphoreType.DMA,
            pltpu.SemaphoreType.DMA,
            pltpu.SemaphoreType.DMA,
            pltpu.SemaphoreType.DMA,
            pltpu.SemaphoreType.DMA,
            pltpu.SemaphoreType.DMA,
            pltpu.VMEM_SHARED((NP, HID), jnp.float32),
        ],
    )(y, row_p, col_p, zeros_rpt)


def _tc_final_body(aggs_ref, y_ref, dinv_ref, b1_ref, batch_ref, w2_ref,
                   b2_ref, out_ref, sums_ref, counts_ref):
    i = pl.program_id(0)

    @pl.when(i == 0)
    def _():
        sums_ref[...] = jnp.zeros_like(sums_ref)
        counts_ref[...] = jnp.zeros_like(counts_ref)

    agg = aggs_ref[0] + aggs_ref[1]
    h = jax.nn.relu(dinv_ref[...] * (agg + y_ref[...]) + b1_ref[...])
    b = batch_ref[0, 0, :]
    gid = lax.broadcasted_iota(jnp.int32, (G, BLK), 0)
    oh = (b[None, :] == gid).astype(jnp.float32)
    sums_ref[...] += jnp.dot(oh, h, preferred_element_type=jnp.float32)
    counts_ref[...] += jnp.broadcast_to(
        jnp.sum(oh, axis=1, keepdims=True), (G, HID))

    @pl.when(i == NBLK - 1)
    def _():
        pooled = sums_ref[...] / jnp.maximum(counts_ref[...], 1.0)
        logits = (jnp.dot(pooled, w2_ref[...],
                          preferred_element_type=jnp.float32) + b2_ref[...])
        lane = lax.broadcasted_iota(jnp.int32, (G, HID), 1)
        valid = lane < OUT
        neg = jnp.full_like(logits, -jnp.inf)
        m = jnp.max(jnp.where(valid, logits, neg), axis=1, keepdims=True)
        e = jnp.where(valid, jnp.exp(logits - m), 0.0)
        lse = jnp.log(jnp.sum(e, axis=1, keepdims=True))
        out_ref[...] = logits - m - lse


def _tc_final(aggs, y, dinv, b1, batch_p, W2p, b2p):
    return pl.pallas_call(
        _tc_final_body,
        grid=(NBLK,),
        in_specs=[
            pl.BlockSpec((NC, BLK, HID), lambda i: (0, i, 0)),
            pl.BlockSpec((BLK, HID), lambda i: (i, 0)),
            pl.BlockSpec((BLK, 1), lambda i: (i, 0)),
            pl.BlockSpec((1, HID), lambda i: (0, 0)),
            pl.BlockSpec((1, 1, BLK), lambda i: (i, 0, 0)),
            pl.BlockSpec((HID, HID), lambda i: (0, 0)),
            pl.BlockSpec((1, HID), lambda i: (0, 0)),
        ],
        out_specs=pl.BlockSpec((G, HID), lambda i: (0, 0)),
        out_shape=jax.ShapeDtypeStruct((G, HID), jnp.float32),
        scratch_shapes=[
            pltpu.VMEM((G, HID), jnp.float32),
            pltpu.VMEM((G, HID), jnp.float32),
        ],
    )(aggs, y, dinv, b1, batch_p, W2p, b2p)


def kernel(x, edge_index, batch, W1, b1, W2, b2):
    pad = E_PAD - E
    idx = jnp.arange(pad, dtype=jnp.int32)
    row = jnp.concatenate([edge_index[0], idx % N])
    col = jnp.concatenate([edge_index[1], N + idx % (NP - N)])
    row_p = row.reshape(TOT_CHUNKS, CHUNK)
    col_p = col.reshape(TOT_CHUNKS, CHUNK)
    x_p = jnp.pad(x, ((0, NP - N), (0, 0)))
    batch_p = jnp.concatenate(
        [batch, jnp.full((NP - N,), G, jnp.int32)]).reshape(NBLK, 1, BLK)
    zeros_rpt = jnp.zeros((RPT, HID), jnp.float32)
    W2p = jnp.pad(W2, ((0, 0), (0, HID - OUT)))
    b2p = jnp.pad(b2, (0, HID - OUT)).reshape(1, HID)
    b1r = b1.reshape(1, HID)

    parts = _sc_degree(edge_index.reshape(-1))
    y, dinv = _tc_scale(x_p, W1, parts)
    aggs = _sc_scatter(y, row_p, col_p, zeros_rpt)
    outp = _tc_final(aggs, y, dinv, b1r, batch_p, W2p, b2p)
    return outp[:, :OUT]
